# Initial kernel scaffold; baseline (speedup 1.0000x reference)
import jax
import jax.numpy as jnp
from jax.experimental import pallas as pl


def kernel(user_id, item_id, user_table, item_table, W1, b1, W2, b2, W3, b3):
    raise NotImplementedError("write your pallas kernel here")


# speedup vs baseline: 2.6277x; 2.6277x over previous
import functools

import jax
import jax.numpy as jnp
from jax import lax
from jax.experimental import pallas as pl
from jax.experimental.pallas import tpu as pltpu
from jax.experimental.pallas import tpu_sc as plsc

B = 16384
D = 128
NC = 2
NS = 16
NW = NC * NS
ROWS_PER_W = B // NW
CH = 128
NCH = ROWS_PER_W // CH

BM = 1024


def _gather_body(user_table, item_table, uid, iid, ue_out, ie_out,
                 idx_u, idx_i, rows, sem):
    wid = lax.axis_index("s") * NC + lax.axis_index("c")
    base = wid * NCH
    pltpu.sync_copy(uid.at[pl.ds(base, NCH)], idx_u)
    pltpu.sync_copy(iid.at[pl.ds(base, NCH)], idx_i)
    cps = [pltpu.async_copy(user_table.at[idx_u.at[j]], rows.at[j], sem)
           for j in range(NCH)]
    for cp in cps:
        cp.wait()
    pltpu.sync_copy(rows, ue_out.at[pl.ds(base, NCH)])
    cps = [pltpu.async_copy(item_table.at[idx_i.at[j]], rows.at[j], sem)
           for j in range(NCH)]
    for cp in cps:
        cp.wait()
    pltpu.sync_copy(rows, ie_out.at[pl.ds(base, NCH)])


@functools.cache
def _sc_gather():
    return pl.kernel(
        _gather_body,
        out_type=(
            jax.ShapeDtypeStruct((B // CH, CH, D), jnp.float32),
            jax.ShapeDtypeStruct((B // CH, CH, D), jnp.float32),
        ),
        mesh=plsc.VectorSubcoreMesh(core_axis_name="c", subcore_axis_name="s"),
        scratch_types=[
            pltpu.VMEM((NCH, CH), jnp.int32),
            pltpu.VMEM((NCH, CH), jnp.int32),
            pltpu.VMEM((NCH, CH, D), jnp.float32),
            pltpu.SemaphoreType.DMA,
        ],
    )


def _mlp_body(ue_ref, ie_ref, w1u_ref, w1i_ref, b1_ref, w2_ref, b2_ref,
              w3_ref, b3_ref, out_ref):
    h1 = jnp.dot(ue_ref[...], w1u_ref[...], preferred_element_type=jnp.float32)
    h1 += jnp.dot(ie_ref[...], w1i_ref[...], preferred_element_type=jnp.float32)
    h1 = jnp.maximum(h1 + b1_ref[...], 0.0)
    h2 = jnp.dot(h1, w2_ref[...], preferred_element_type=jnp.float32)
    h2 = jnp.maximum(h2 + b2_ref[...], 0.0)
    out_ref[...] = jnp.sum(h2 * w3_ref[...], axis=1) + b3_ref[0, 0]


def _mlp(ue, ie, W1u, W1i, b1, W2, b2, w3t, b3):
    return pl.pallas_call(
        _mlp_body,
        grid=(B // BM,),
        in_specs=[
            pl.BlockSpec((BM, D), lambda i: (i, 0)),
            pl.BlockSpec((BM, D), lambda i: (i, 0)),
            pl.BlockSpec((D, 128), lambda i: (0, 0)),
            pl.BlockSpec((D, 128), lambda i: (0, 0)),
            pl.BlockSpec((1, 128), lambda i: (0, 0)),
            pl.BlockSpec((128, 64), lambda i: (0, 0)),
            pl.BlockSpec((1, 64), lambda i: (0, 0)),
            pl.BlockSpec((1, 64), lambda i: (0, 0)),
            pl.BlockSpec((1, 1), lambda i: (0, 0)),
        ],
        out_specs=pl.BlockSpec((BM,), lambda i: (i,)),
        out_shape=jax.ShapeDtypeStruct((B,), jnp.float32),
    )(ue, ie, W1u, W1i, b1, W2, b2, w3t, b3)


def kernel(user_id, item_id, user_table, item_table, W1, b1, W2, b2, W3, b3):
    uid = user_id.astype(jnp.int32).reshape(B // CH, CH)
    iid = item_id.astype(jnp.int32).reshape(B // CH, CH)
    ue3, ie3 = _sc_gather()(user_table, item_table, uid, iid)
    ue = ue3.reshape(B, D)
    ie = ie3.reshape(B, D)
    W1u = W1[:D]
    W1i = W1[D:]
    return _mlp(ue, ie, W1u, W1i, b1.reshape(1, 128), W2, b2.reshape(1, 64),
                W3.reshape(1, 64), b3.reshape(1, 1))


# speedup vs baseline: 2.9892x; 1.1376x over previous
import functools

import jax
import jax.numpy as jnp
from jax import lax
from jax.experimental import pallas as pl
from jax.experimental.pallas import tpu as pltpu
from jax.experimental.pallas import tpu_sc as plsc

B = 16384
D = 128
NC = 2
NS = 16
NW = NC * NS
ROWS_PER_W = B // NW
CH = 128
NCH = ROWS_PER_W // CH

BM = 2048


def _gather_body(user_table, item_table, uid, iid, ue_out, ie_out,
                 idx_u, idx_i, rows, sem):
    wid = lax.axis_index("s") * NC + lax.axis_index("c")
    base = wid * NCH
    pltpu.sync_copy(uid.at[pl.ds(base, NCH)], idx_u)
    pltpu.sync_copy(iid.at[pl.ds(base, NCH)], idx_i)
    cps = [pltpu.async_copy(user_table.at[idx_u.at[j]], rows.at[j], sem)
           for j in range(NCH)]
    for cp in cps:
        cp.wait()
    pltpu.sync_copy(rows, ue_out.at[pl.ds(base, NCH)])
    cps = [pltpu.async_copy(item_table.at[idx_i.at[j]], rows.at[j], sem)
           for j in range(NCH)]
    for cp in cps:
        cp.wait()
    pltpu.sync_copy(rows, ie_out.at[pl.ds(base, NCH)])


@functools.cache
def _sc_gather():
    return pl.kernel(
        _gather_body,
        out_type=(
            jax.ShapeDtypeStruct((B // CH, CH, D), jnp.float32),
            jax.ShapeDtypeStruct((B // CH, CH, D), jnp.float32),
        ),
        mesh=plsc.VectorSubcoreMesh(core_axis_name="c", subcore_axis_name="s"),
        scratch_types=[
            pltpu.VMEM((NCH, CH), jnp.int32),
            pltpu.VMEM((NCH, CH), jnp.int32),
            pltpu.VMEM((NCH, CH, D), jnp.float32),
            pltpu.SemaphoreType.DMA,
        ],
    )


def _mlp_body(ue_ref, ie_ref, w1u_ref, w1i_ref, b1_ref, w2_ref, b2_ref,
              w3_ref, b3_ref, out_ref):
    h1 = jnp.dot(ue_ref[...], w1u_ref[...], preferred_element_type=jnp.float32)
    h1 += jnp.dot(ie_ref[...], w1i_ref[...], preferred_element_type=jnp.float32)
    h1 = jnp.maximum(h1 + b1_ref[...], 0.0)
    h2 = jnp.dot(h1, w2_ref[...], preferred_element_type=jnp.float32)
    h2 = jnp.maximum(h2 + b2_ref[...], 0.0)
    out_ref[...] = jnp.sum(h2 * w3_ref[...], axis=1, keepdims=True) + b3_ref[0, 0]


def _mlp(ue, ie, W1u, W1i, b1, W2, b2, w3t, b3):
    return pl.pallas_call(
        _mlp_body,
        grid=(B // BM,),
        in_specs=[
            pl.BlockSpec((BM, D), lambda i: (i, 0)),
            pl.BlockSpec((BM, D), lambda i: (i, 0)),
            pl.BlockSpec((D, 128), lambda i: (0, 0)),
            pl.BlockSpec((D, 128), lambda i: (0, 0)),
            pl.BlockSpec((1, 128), lambda i: (0, 0)),
            pl.BlockSpec((128, 64), lambda i: (0, 0)),
            pl.BlockSpec((1, 64), lambda i: (0, 0)),
            pl.BlockSpec((1, 64), lambda i: (0, 0)),
            pl.BlockSpec((1, 1), lambda i: (0, 0)),
        ],
        out_specs=pl.BlockSpec((BM, 1), lambda i: (i, 0)),
        out_shape=jax.ShapeDtypeStruct((B, 1), jnp.float32),
    )(ue, ie, W1u, W1i, b1, W2, b2, w3t, b3)


def kernel(user_id, item_id, user_table, item_table, W1, b1, W2, b2, W3, b3):
    uid = user_id.astype(jnp.int32).reshape(B // CH, CH)
    iid = item_id.astype(jnp.int32).reshape(B // CH, CH)
    ue3, ie3 = _sc_gather()(user_table, item_table, uid, iid)
    ue = ue3.reshape(B, D)
    ie = ie3.reshape(B, D)
    W1u = W1[:D]
    W1i = W1[D:]
    out = _mlp(ue, ie, W1u, W1i, b1.reshape(1, 128), W2, b2.reshape(1, 64),
               W3.reshape(1, 64), b3.reshape(1, 1))
    return out.reshape(B)


# speedup vs baseline: 3.3198x; 1.1106x over previous
import functools

import jax
import jax.numpy as jnp
from jax import lax
from jax.experimental import pallas as pl
from jax.experimental.pallas import tpu as pltpu
from jax.experimental.pallas import tpu_sc as plsc

B = 16384
D = 128
NC = 2
NS = 16
NW = NC * NS
ROWS_PER_W = B // NW
CH = 128
NCH = ROWS_PER_W // CH
NB = 7

BM = 2048


def _gather_body(user_table, item_table, uid, iid, ue_out, ie_out,
                 idx, bufs, gsem, csem):
    wid = lax.axis_index("s") * NC + lax.axis_index("c")
    rbase = wid * ROWS_PER_W
    cbase = wid * NCH
    pltpu.sync_copy(uid.at[pl.ds(rbase, ROWS_PER_W)],
                    idx.at[pl.ds(0, ROWS_PER_W)])
    pltpu.sync_copy(iid.at[pl.ds(rbase, ROWS_PER_W)],
                    idx.at[pl.ds(ROWS_PER_W, ROWS_PER_W)])
    tables = [user_table] * NCH + [item_table] * NCH
    outs = [ue_out] * NCH + [ie_out] * NCH
    nch2 = 2 * NCH
    g = [None] * nch2
    c = [None] * nch2
    for k in range(nch2):
        if k >= NB:
            c[k - NB].wait()
        g[k] = pltpu.async_copy(
            tables[k].at[idx.at[pl.ds(k * CH, CH)]], bufs.at[k % NB], gsem)
        if k >= 1:
            g[k - 1].wait()
            c[k - 1] = pltpu.async_copy(
                bufs.at[(k - 1) % NB],
                outs[k - 1].at[cbase + ((k - 1) % NCH)], csem)
    g[nch2 - 1].wait()
    c[nch2 - 1] = pltpu.async_copy(
        bufs.at[(nch2 - 1) % NB], outs[nch2 - 1].at[cbase + NCH - 1], csem)
    for k in range(1, nch2):
        c[k].wait()


@functools.cache
def _sc_gather():
    return pl.kernel(
        _gather_body,
        out_type=(
            jax.ShapeDtypeStruct((B // CH, CH, D), jnp.float32),
            jax.ShapeDtypeStruct((B // CH, CH, D), jnp.float32),
        ),
        mesh=plsc.VectorSubcoreMesh(core_axis_name="c", subcore_axis_name="s"),
        scratch_types=[
            pltpu.VMEM((2 * ROWS_PER_W,), jnp.int32),
            pltpu.VMEM((NB, CH, D), jnp.float32),
            pltpu.SemaphoreType.DMA,
            pltpu.SemaphoreType.DMA,
        ],
    )


def _mlp_body(ue_ref, ie_ref, w1u_ref, w1i_ref, b1_ref, w2_ref, b2_ref,
              w3_ref, b3_ref, out_ref):
    h1 = jnp.dot(ue_ref[...], w1u_ref[...], preferred_element_type=jnp.float32)
    h1 += jnp.dot(ie_ref[...], w1i_ref[...], preferred_element_type=jnp.float32)
    h1 = jnp.maximum(h1 + b1_ref[...], 0.0)
    h2 = jnp.dot(h1, w2_ref[...], preferred_element_type=jnp.float32)
    h2 = jnp.maximum(h2 + b2_ref[...], 0.0)
    r = lax.dot_general(w3_ref[...], h2, (((1,), (1,)), ((), ())),
                        preferred_element_type=jnp.float32) + b3_ref[0, 0]
    out_ref[...] = r.reshape(1, 1, r.shape[1])


def _mlp(ue, ie, W1a, W1b, b1, W2, b2, w3t, b3):
    return pl.pallas_call(
        _mlp_body,
        grid=(B // BM,),
        in_specs=[
            pl.BlockSpec((BM, D), lambda i: (i, 0)),
            pl.BlockSpec((BM, D), lambda i: (i, 0)),
            pl.BlockSpec((D, 128), lambda i: (0, 0)),
            pl.BlockSpec((D, 128), lambda i: (1, 0)),
            pl.BlockSpec((1, 128), lambda i: (0, 0)),
            pl.BlockSpec((128, 64), lambda i: (0, 0)),
            pl.BlockSpec((1, 64), lambda i: (0, 0)),
            pl.BlockSpec((1, 64), lambda i: (0, 0)),
            pl.BlockSpec((1, 1), lambda i: (0, 0)),
        ],
        out_specs=pl.BlockSpec((1, 1, BM), lambda i: (i, 0, 0)),
        out_shape=jax.ShapeDtypeStruct((B // BM, 1, BM), jnp.float32),
    )(ue, ie, W1a, W1b, b1, W2, b2, w3t, b3)


def kernel(user_id, item_id, user_table, item_table, W1, b1, W2, b2, W3, b3):
    uid = user_id.astype(jnp.int32)
    iid = item_id.astype(jnp.int32)
    ue3, ie3 = _sc_gather()(user_table, item_table, uid, iid)
    ue = ue3.reshape(B, D)
    ie = ie3.reshape(B, D)
    out = _mlp(ue, ie, W1, W1, b1.reshape(1, 128), W2, b2.reshape(1, 64),
               W3.reshape(1, 64), b3.reshape(1, 1))
    return out.reshape(B)


# speedup vs baseline: 3.3587x; 1.0117x over previous
import functools

import jax
import jax.numpy as jnp
from jax import lax
from jax.experimental import pallas as pl
from jax.experimental.pallas import tpu as pltpu
from jax.experimental.pallas import tpu_sc as plsc

B = 16384
D = 128
NC = 2
NS = 16
NW = NC * NS
ROWS_PER_W = B // NW
CH = 128
NCH = ROWS_PER_W // CH
NB = 7

BM = 2048


def _gather_body(user_table, item_table, uid, iid, ue_out, ie_out,
                 idx_u, idx_i, rows, sem):
    wid = lax.axis_index("s") * NC + lax.axis_index("c")
    rbase = wid * ROWS_PER_W
    pltpu.sync_copy(uid.at[pl.ds(rbase, ROWS_PER_W)], idx_u)
    pltpu.sync_copy(iid.at[pl.ds(rbase, ROWS_PER_W)], idx_i)
    pltpu.async_copy(user_table.at[idx_u], rows, sem).wait()
    pltpu.sync_copy(rows, ue_out.at[pl.ds(rbase, ROWS_PER_W)])
    pltpu.async_copy(item_table.at[idx_i], rows, sem).wait()
    pltpu.sync_copy(rows, ie_out.at[pl.ds(rbase, ROWS_PER_W)])


@functools.cache
def _sc_gather():
    return pl.kernel(
        _gather_body,
        out_type=(
            jax.ShapeDtypeStruct((B, D), jnp.float32),
            jax.ShapeDtypeStruct((B, D), jnp.float32),
        ),
        mesh=plsc.VectorSubcoreMesh(core_axis_name="c", subcore_axis_name="s"),
        scratch_types=[
            pltpu.VMEM((ROWS_PER_W,), jnp.int32),
            pltpu.VMEM((ROWS_PER_W,), jnp.int32),
            pltpu.VMEM((ROWS_PER_W, D), jnp.float32),
            pltpu.SemaphoreType.DMA,
        ],
    )


def _mlp_body(ue_ref, ie_ref, w1u_ref, w1i_ref, b1_ref, w2_ref, b2_ref,
              w3_ref, b3_ref, out_ref):
    h1 = jnp.dot(ue_ref[...], w1u_ref[...], preferred_element_type=jnp.float32)
    h1 += jnp.dot(ie_ref[...], w1i_ref[...], preferred_element_type=jnp.float32)
    h1 = jnp.maximum(h1 + b1_ref[...], 0.0)
    h2 = jnp.dot(h1, w2_ref[...], preferred_element_type=jnp.float32)
    h2 = jnp.maximum(h2 + b2_ref[...], 0.0)
    r = lax.dot_general(w3_ref[...], h2, (((1,), (1,)), ((), ())),
                        preferred_element_type=jnp.float32) + b3_ref[0, 0]
    out_ref[...] = r.reshape(1, 1, r.shape[1])


def _mlp(ue, ie, W1a, W1b, b1, W2, b2, w3t, b3):
    return pl.pallas_call(
        _mlp_body,
        grid=(B // BM,),
        in_specs=[
            pl.BlockSpec((BM, D), lambda i: (i, 0)),
            pl.BlockSpec((BM, D), lambda i: (i, 0)),
            pl.BlockSpec((D, 128), lambda i: (0, 0)),
            pl.BlockSpec((D, 128), lambda i: (1, 0)),
            pl.BlockSpec((1, 128), lambda i: (0, 0)),
            pl.BlockSpec((128, 64), lambda i: (0, 0)),
            pl.BlockSpec((1, 64), lambda i: (0, 0)),
            pl.BlockSpec((1, 64), lambda i: (0, 0)),
            pl.BlockSpec((1, 1), lambda i: (0, 0)),
        ],
        out_specs=pl.BlockSpec((1, 1, BM), lambda i: (i, 0, 0)),
        out_shape=jax.ShapeDtypeStruct((B // BM, 1, BM), jnp.float32),
    )(ue, ie, W1a, W1b, b1, W2, b2, w3t, b3)


def kernel(user_id, item_id, user_table, item_table, W1, b1, W2, b2, W3, b3):
    ue, ie = _sc_gather()(user_table, item_table, user_id, item_id)
    out = _mlp(ue, ie, W1, W1, b1.reshape(1, 128), W2, b2.reshape(1, 64),
               W3.reshape(1, 64), b3.reshape(1, 1))
    return out.reshape(B)
